# baseline (speedup 1.0000x reference)
import jax
import jax.numpy as jnp
from jax.experimental import pallas as pl

EMBED = 256
NH = 8
NL = 4
NP = 8
_SS = ((92, 160), (46, 80), (23, 40), (12, 20))


def _mm_bias(x, w, b, block_m=512):
    M, K = x.shape
    N = w.shape[1]
    Mp = ((M + block_m - 1) // block_m) * block_m
    xp = jnp.pad(x, ((0, Mp - M), (0, 0))) if Mp != M else x

    def body(x_ref, w_ref, b_ref, o_ref):
        o_ref[...] = jnp.dot(x_ref[...], w_ref[...],
                             preferred_element_type=jnp.float32) + b_ref[...]

    out = pl.pallas_call(
        body,
        grid=(Mp // block_m,),
        in_specs=[pl.BlockSpec((block_m, K), lambda i: (i, 0)),
                  pl.BlockSpec((K, N), lambda i: (0, 0)),
                  pl.BlockSpec((1, N), lambda i: (0, 0))],
        out_specs=pl.BlockSpec((block_m, N), lambda i: (i, 0)),
        out_shape=jax.ShapeDtypeStruct((Mp, N), jnp.float32),
    )(xp, w, b[None, :])
    return out[:M]


def _sample(value, loc, aw):
    bs, nv, nh, dh = value.shape
    nq = loc.shape[1]
    npnt = loc.shape[4]
    out = jnp.zeros((bs, nh, nq, dh), dtype=value.dtype)
    start = 0
    for l in range(NL):
        H, W = _SS[l]
        vl = value[:, start:start + H * W].transpose(0, 2, 1, 3)
        start += H * W
        ll = loc[:, :, :, l]
        x = ll[..., 0] * W - 0.5
        y = ll[..., 1] * H - 0.5
        x0 = jnp.floor(x)
        y0 = jnp.floor(y)
        tx = x - x0
        ty = y - y0
        al = aw[:, :, :, l].transpose(0, 2, 1, 3)
        for cx, wx in ((x0, 1.0 - tx), (x0 + 1.0, tx)):
            for cy, wy in ((y0, 1.0 - ty), (y0 + 1.0, ty)):
                valid = ((cx >= 0) & (cx < W) & (cy >= 0) & (cy < H)).astype(value.dtype)
                xi = jnp.clip(cx, 0, W - 1).astype(jnp.int32)
                yi = jnp.clip(cy, 0, H - 1).astype(jnp.int32)
                idx = (yi * W + xi).transpose(0, 2, 1, 3).reshape(bs, nh, nq * npnt)
                g = jnp.take_along_axis(vl, idx[..., None], axis=2).reshape(bs, nh, nq, npnt, dh)
                w = (wx * wy).transpose(0, 2, 1, 3) * valid.transpose(0, 2, 1, 3) * al
                out = out + jnp.einsum('bhqp,bhqpd->bhqd', w, g)
    return out.transpose(0, 2, 1, 3).reshape(bs, nq, nh * dh)


def kernel(query, value, reference_points, spatial_shapes, level_start_index,
           W_off, b_off, W_attn, b_attn, W_val, b_val, W_out, b_out):
    bs, nq, d = query.shape
    nv = value.shape[1]
    dh = d // NH

    v = _mm_bias(value.reshape(bs * nv, d), W_val, b_val).reshape(bs, nv, NH, dh)

    qw = jnp.concatenate([W_off, W_attn], axis=1)
    qb = jnp.concatenate([b_off, b_attn], axis=0)
    qproj = _mm_bias(query.reshape(bs * nq, d), qw, qb)
    off = qproj[:, :NH * NL * NP * 2].reshape(bs, nq, NH, NL, NP, 2)
    aw = jax.nn.softmax(qproj[:, NH * NL * NP * 2:].reshape(bs, nq, NH, NL * NP), axis=-1)
    aw = aw.reshape(bs, nq, NH, NL, NP)

    ss_f = spatial_shapes.astype(jnp.float32)
    norm = jnp.stack([ss_f[:, 1], ss_f[:, 0]], axis=-1)
    nZ = reference_points.shape[2]
    ref = reference_points[:, :, None, None, None, :, :]
    off_n = off / norm[None, None, None, :, None, :]
    off_n = off_n.reshape(bs, nq, NH, NL, NP // nZ, nZ, 2)
    loc = (ref + off_n).reshape(bs, nq, NH, NL, NP, 2)

    res = _sample(v, loc, aw)
    return _mm_bias(res.reshape(bs * nq, d), W_out, b_out).reshape(bs, nq, d)


# speedup vs baseline: 22.8472x; 22.8472x over previous
import functools

import jax
import jax.numpy as jnp
from jax import lax
from jax.experimental import pallas as pl
from jax.experimental.pallas import tpu as pltpu
from jax.experimental.pallas import tpu_sc as plsc

EMBED = 256
NH = 8
NL = 4
NP = 8
DH = 32
_SS = ((92, 160), (46, 80), (23, 40), (12, 20))
_LVL_BASE = (0, 14720, 18400, 19320)
NV = 19560

BS = 2
NQ = 2048
ROWS = BS * NQ * NH
GPR = NL * NP * 4
NWORK = 32
RPW = ROWS // NWORK
BLK = 8
NBLK = RPW // BLK
LANES = 16


def _mm_bias(x, w, b, block_m=512):
    M, K = x.shape
    N = w.shape[1]
    Mp = ((M + block_m - 1) // block_m) * block_m
    xp = jnp.pad(x, ((0, Mp - M), (0, 0))) if Mp != M else x

    def body(x_ref, w_ref, b_ref, o_ref):
        o_ref[...] = jnp.dot(x_ref[...], w_ref[...],
                             preferred_element_type=jnp.float32) + b_ref[...]

    out = pl.pallas_call(
        body,
        grid=(Mp // block_m,),
        in_specs=[pl.BlockSpec((block_m, K), lambda i: (i, 0)),
                  pl.BlockSpec((K, N), lambda i: (0, 0)),
                  pl.BlockSpec((1, N), lambda i: (0, 0))],
        out_specs=pl.BlockSpec((block_m, N), lambda i: (i, 0)),
        out_shape=jax.ShapeDtypeStruct((Mp, N), jnp.float32),
    )(xp, w, b[None, :])
    return out[:M]


def _sc_gather_reduce(vt, idx, wts):
    mesh = plsc.VectorSubcoreMesh(core_axis_name="c", subcore_axis_name="s")

    @functools.partial(
        pl.kernel,
        out_type=jax.ShapeDtypeStruct((ROWS, DH), jnp.float32),
        mesh=mesh,
        compiler_params=pltpu.CompilerParams(use_tc_tiling_on_sc=False),
        scratch_types=[
            pltpu.VMEM((BLK, GPR), jnp.int32),
            pltpu.VMEM((BLK * GPR,), jnp.float32),
            pltpu.VMEM((BLK, GPR, DH), jnp.float32),
            pltpu.VMEM((BLK, DH), jnp.float32),
            pltpu.SemaphoreType.DMA,
        ],
    )
    def body(vt_hbm, idx_hbm, w_hbm, out_hbm, idx_v, w_v, rows_v, outb, sem):
        wid = lax.axis_index("s") * 2 + lax.axis_index("c")
        base = wid * RPW
        splats = [jnp.full((LANES, 1), t, jnp.int32) for t in range(LANES)]
        gdn = lax.GatherDimensionNumbers(
            offset_dims=(), collapsed_slice_dims=(0,), start_index_map=(0,))

        def bcast(vec, t):
            return lax.gather(vec, splats[t], gdn, (1,),
                              mode=lax.GatherScatterMode.PROMISE_IN_BOUNDS)

        def blk_body(i, carry):
            r0 = base + i * BLK
            pltpu.sync_copy(idx_hbm.at[pl.ds(r0, BLK)], idx_v)
            pltpu.sync_copy(w_hbm.at[pl.ds(r0 * GPR, BLK * GPR)], w_v)
            copies = [
                pltpu.async_copy(vt_hbm.at[idx_v.at[r]], rows_v.at[r], sem)
                for r in range(BLK)
            ]
            for cp in copies:
                cp.wait()

            def row_body(r, carry2):
                def chunk(c, acc):
                    a0, a1 = acc
                    wchunk = w_v[pl.ds(r * GPR + c * LANES, LANES)]
                    for t in range(LANES):
                        wb = bcast(wchunk, t)
                        j = c * LANES + t
                        lo = rows_v[r, j, pl.ds(0, LANES)]
                        hi = rows_v[r, j, pl.ds(LANES, LANES)]
                        a0 = a0 + wb * lo
                        a1 = a1 + wb * hi
                    return a0, a1

                z = jnp.zeros((LANES,), jnp.float32)
                a0, a1 = lax.fori_loop(0, GPR // LANES, chunk, (z, z))
                outb[r, pl.ds(0, LANES)] = a0
                outb[r, pl.ds(LANES, LANES)] = a1
                return carry2

            lax.fori_loop(0, BLK, row_body, 0)
            pltpu.sync_copy(outb, out_hbm.at[pl.ds(r0, BLK)])
            return carry

        lax.fori_loop(0, NBLK, blk_body, 0)

    return body(vt, idx, wts)


def kernel(query, value, reference_points, spatial_shapes, level_start_index,
           W_off, b_off, W_attn, b_attn, W_val, b_val, W_out, b_out):
    bs, nq, d = query.shape
    nv = value.shape[1]

    v = _mm_bias(value.reshape(bs * nv, d), W_val, b_val)
    vt = v.reshape(bs, nv, NH, DH).transpose(0, 2, 1, 3).reshape(bs * NH * nv, DH)

    qw = jnp.concatenate([W_off, W_attn], axis=1)
    qb = jnp.concatenate([b_off, b_attn], axis=0)
    qproj = _mm_bias(query.reshape(bs * nq, d), qw, qb)
    off = qproj[:, :NH * NL * NP * 2].reshape(bs, nq, NH, NL, NP, 2)
    aw = jax.nn.softmax(
        qproj[:, NH * NL * NP * 2:].reshape(bs, nq, NH, NL * NP), axis=-1)
    aw = aw.reshape(bs, nq, NH, NL, NP)

    ss_f = spatial_shapes.astype(jnp.float32)
    norm = jnp.stack([ss_f[:, 1], ss_f[:, 0]], axis=-1)
    nZ = reference_points.shape[2]
    ref = reference_points[:, :, None, None, None, :, :]
    off_n = off / norm[None, None, None, :, None, :]
    off_n = off_n.reshape(bs, nq, NH, NL, NP // nZ, nZ, 2)
    loc = (ref + off_n).reshape(bs, nq, NH, NL, NP, 2)

    Wl = jnp.array([s[1] for s in _SS], jnp.float32)[:, None]
    Hl = jnp.array([s[0] for s in _SS], jnp.float32)[:, None]
    x = loc[..., 0] * Wl - 0.5
    y = loc[..., 1] * Hl - 0.5
    x0 = jnp.floor(x)
    y0 = jnp.floor(y)
    tx = x - x0
    ty = y - y0
    lvl_base = jnp.array(_LVL_BASE, jnp.int32)[:, None]
    bh_base = ((jnp.arange(bs, dtype=jnp.int32)[:, None] * NH
                + jnp.arange(NH, dtype=jnp.int32)[None, :]) * nv)
    bh_base = bh_base[:, None, :, None, None]

    idx_c = []
    w_c = []
    for cx, wx in ((x0, 1.0 - tx), (x0 + 1.0, tx)):
        for cy, wy in ((y0, 1.0 - ty), (y0 + 1.0, ty)):
            valid = ((cx >= 0) & (cx < Wl) & (cy >= 0) & (cy < Hl)
                     ).astype(jnp.float32)
            xi = jnp.clip(cx, 0, Wl - 1).astype(jnp.int32)
            yi = jnp.clip(cy, 0, Hl - 1).astype(jnp.int32)
            idx_c.append(yi * Wl.astype(jnp.int32) + xi + lvl_base + bh_base)
            w_c.append(wx * wy * valid * aw)
    idx = jnp.stack(idx_c, axis=-1).reshape(ROWS, GPR)
    wts = jnp.stack(w_c, axis=-1).reshape(ROWS * GPR)

    res = _sc_gather_reduce(vt, idx, wts)

    out = _mm_bias(res.reshape(bs * nq, d), W_out, b_out)
    return out.reshape(bs, nq, d)
